# Initial kernel scaffold; baseline (speedup 1.0000x reference)
import jax
import jax.numpy as jnp
from jax.experimental import pallas as pl


def kernel(x, edge_index, edge_attr, W, att_src, att_dst, W_edge, att_edge):
    raise NotImplementedError("write your pallas kernel here")


# speedup vs baseline: 11.2774x; 11.2774x over previous
import functools

import jax
import jax.numpy as jnp
from jax import lax
from jax.experimental import pallas as pl
from jax.experimental.pallas import tpu as pltpu
from jax.experimental.pallas import tpu_sc as plsc

N = 10000
E = 320000
D = 128
NEG = 0.2

NC = 2
NS = 16
NW = NC * NS
CPT = E // NW
BB = 80
NB = CPT // BB
NP = 10240
STRIPE = NP // NS


def _dense_body(x_ref, w_ref, asrc_ref, adst_ref, ea2_ref, m_ref,
                h_ref, ansrc_ref, andst_ref, ae2_ref, aes_ref):
    h = jnp.dot(x_ref[...], w_ref[...], preferred_element_type=jnp.float32)
    h_ref[...] = h
    ansrc_ref[...] = jnp.sum(h * asrc_ref[...], axis=1, keepdims=True)
    andst_ref[...] = jnp.sum(h * adst_ref[...], axis=1, keepdims=True)
    ae2 = jnp.dot(ea2_ref[...], m_ref[...], preferred_element_type=jnp.float32)
    ae2_ref[...] = ae2
    aes_ref[...] = jnp.sum(ae2).reshape(1, 1)


_dense_call = pl.pallas_call(
    _dense_body,
    out_shape=[
        jax.ShapeDtypeStruct((N, D), jnp.float32),
        jax.ShapeDtypeStruct((N, 1), jnp.float32),
        jax.ShapeDtypeStruct((N, 1), jnp.float32),
        jax.ShapeDtypeStruct((E // D, D), jnp.float32),
        jax.ShapeDtypeStruct((1, 1), jnp.float32),
    ],
)


_mesh = plsc.VectorSubcoreMesh(core_axis_name="c", subcore_axis_name="s")


DC = D // 2


@functools.partial(
    pl.kernel,
    out_type=[
        jax.ShapeDtypeStruct((NC, 2, NP, DC), jnp.float32),
        jax.ShapeDtypeStruct((NC, NP, 16), jnp.float32),
    ],
    mesh=_mesh,
    compiler_params=pltpu.CompilerParams(needs_layout_passes=False,
                                         use_tc_tiling_on_sc=False),
    scratch_types=[
        pltpu.VMEM((NB, BB), jnp.int32),
        pltpu.VMEM((NB, BB), jnp.int32),
        pltpu.VMEM((NB, BB), jnp.float32),
        pltpu.VMEM((N,), jnp.float32),
        pltpu.VMEM((N,), jnp.float32),
        pltpu.VMEM((BB, DC), jnp.float32),
        pltpu.VMEM((BB, 16), jnp.float32),
        pltpu.VMEM((128, DC), jnp.float32),
        pltpu.VMEM((128, 16), jnp.float32),
        pltpu.VMEM_SHARED((NP, DC), jnp.float32),
        pltpu.VMEM_SHARED((NP, 16), jnp.float32),
        pltpu.SemaphoreType.DMA,
    ],
)
def _sc_edges(h0_hbm, h1_hbm, asrc_hbm, adst_hbm, src_hbm, dst_hbm, ae_hbm,
              msg_out, den_out,
              src_v, dst_v, p_v, asrc_t, adst_t, rows, prows,
              zmsg, zden, accm, accd, sem):
    cid = lax.axis_index("c")
    sid = lax.axis_index("s")
    wid = sid * NC + cid

    pltpu.sync_copy(src_hbm.at[wid], src_v)
    pltpu.sync_copy(dst_hbm.at[wid], dst_v)
    pltpu.sync_copy(ae_hbm.at[wid], p_v)
    pltpu.sync_copy(asrc_hbm, asrc_t)
    pltpu.sync_copy(adst_hbm, adst_t)

    z16 = jnp.zeros((16,), jnp.float32)

    @pl.loop(0, 128)
    def _zrow(r):
        for c in range(DC // 16):
            zmsg[r, pl.ds(c * 16, 16)] = z16
        zden[r, :] = z16

    for k in range(STRIPE // 128):
        off = sid * STRIPE + k * 128
        pltpu.sync_copy(zmsg, accm.at[pl.ds(off, 128)])
        pltpu.sync_copy(zden, accd.at[pl.ds(off, 128)])

    @pl.loop(0, NB)
    def _alpha(b):
        for k in range(BB // 16):
            sl = pl.ds(k * 16, 16)
            s16 = src_v[b, sl]
            d16 = dst_v[b, sl]
            al = (plsc.load_gather(asrc_t, [s16])
                  + plsc.load_gather(adst_t, [d16])
                  + p_v[b, sl])
            al = jnp.where(al >= 0.0, al, al * NEG)
            p_v[b, sl] = jnp.exp(al)

    plsc.subcore_barrier()

    for half, h_hbm in ((0, h0_hbm), (1, h1_hbm)):
        @pl.loop(0, NB)
        def _batch(b):
            pltpu.async_copy(h_hbm.at[src_v.at[b]], rows, sem).wait()

            @pl.loop(0, BB // 16)
            def _grp(g):
                p16 = p_v[b, pl.ds(g * 16, 16)]
                for j in range(16):
                    r = g * 16 + j
                    pb = jnp.full((16,), p16[j], dtype=jnp.float32)
                    if half == 0:
                        prows[r, :] = pb
                    for c in range(DC // 16):
                        sl = pl.ds(c * 16, 16)
                        rows[r, sl] = rows[r, sl] * pb

            pltpu.sync_copy(rows, accm.at[dst_v.at[b]], add=True)
            if half == 0:
                pltpu.sync_copy(prows, accd.at[dst_v.at[b]], add=True)

        plsc.subcore_barrier()

        off = sid * STRIPE
        pltpu.sync_copy(accm.at[pl.ds(off, STRIPE)],
                        msg_out.at[cid, half, pl.ds(off, STRIPE)])
        if half == 0:
            pltpu.sync_copy(accd.at[pl.ds(off, STRIPE)],
                            den_out.at[cid, pl.ds(off, STRIPE)])
            for k in range(STRIPE // 128):
                zoff = sid * STRIPE + k * 128
                pltpu.sync_copy(zmsg, accm.at[pl.ds(zoff, 128)])
            plsc.subcore_barrier()


def _combine_body(mp_ref, dp_ref, h_ref, ansrc_ref, andst_ref, aself_ref,
                  out_ref):
    al = ansrc_ref[...] + andst_ref[...] + aself_ref[...]
    al = jnp.where(al >= 0.0, al, al * NEG)
    ps = jnp.exp(al)
    den = dp_ref[0, :, 0:1] + dp_ref[1, :, 0:1] + ps + 1e-16
    msg = jnp.concatenate(
        [mp_ref[0, 0] + mp_ref[1, 0],
         mp_ref[0, 1] + mp_ref[1, 1]], axis=1)
    out_ref[...] = (msg + ps * h_ref[...]) / den


BN = 1000

_combine_call = pl.pallas_call(
    _combine_body,
    grid=(N // BN,),
    in_specs=[
        pl.BlockSpec((NC, 2, BN, DC), lambda i: (0, 0, i, 0)),
        pl.BlockSpec((NC, BN, 16), lambda i: (0, i, 0)),
        pl.BlockSpec((BN, D), lambda i: (i, 0)),
        pl.BlockSpec((BN, 1), lambda i: (i, 0)),
        pl.BlockSpec((BN, 1), lambda i: (i, 0)),
        pl.BlockSpec((1, 1), lambda i: (0, 0)),
    ],
    out_specs=pl.BlockSpec((BN, D), lambda i: (i, 0)),
    out_shape=jax.ShapeDtypeStruct((N, D), jnp.float32),
)


@jax.jit
def kernel(x, edge_index, edge_attr, W, att_src, att_dst, W_edge, att_edge):
    we = W_edge @ att_edge
    we16 = jnp.pad(we, (0, 5))
    cidx = jnp.arange(16 * D, dtype=jnp.int32)
    m = jnp.zeros((16 * D, D), jnp.float32).at[cidx, cidx // 16].set(we16[cidx % 16])

    ea2 = jnp.pad(edge_attr, ((0, 0), (0, 5))).reshape(E // D, 16 * D)

    h, ansrc, andst, ae2, aes = _dense_call(
        x, W, att_src.reshape(1, D), att_dst.reshape(1, D), ea2, m)

    src = edge_index[0].reshape(NW, NB, BB)
    dst = edge_index[1].reshape(NW, NB, BB)
    ae = ae2.reshape(NW, NB, BB)

    h0 = h[:, :DC]
    h1 = h[:, DC:]
    msg, den = _sc_edges(h0, h1, ansrc.reshape(N), andst.reshape(N),
                         src, dst, ae)

    return _combine_call(msg, den, h, ansrc, andst, aes / E)


# speedup vs baseline: 12.1651x; 1.0787x over previous
import functools

import jax
import jax.numpy as jnp
from jax import lax
from jax.experimental import pallas as pl
from jax.experimental.pallas import tpu as pltpu
from jax.experimental.pallas import tpu_sc as plsc

N = 10000
E = 320000
D = 128
NEG = 0.2

NC = 2
NS = 16
NW = NC * NS
CPT = E // NW
BB = 80
NB = CPT // BB
NP = 10240
STRIPE = NP // NS


def _dense_body(x_ref, w_ref, asrc_ref, adst_ref, ea2_ref, m_ref,
                h_ref, ansrc_ref, andst_ref, ae2_ref, aes_ref):
    h = jnp.dot(x_ref[...], w_ref[...], preferred_element_type=jnp.float32)
    h_ref[...] = h
    ansrc_ref[...] = jnp.sum(h * asrc_ref[...], axis=1, keepdims=True)
    andst_ref[...] = jnp.sum(h * adst_ref[...], axis=1, keepdims=True)
    ae2 = jnp.dot(ea2_ref[...], m_ref[...], preferred_element_type=jnp.float32)
    ae2_ref[...] = ae2
    aes_ref[...] = jnp.sum(ae2).reshape(1, 1)


_dense_call = pl.pallas_call(
    _dense_body,
    out_shape=[
        jax.ShapeDtypeStruct((N, D), jnp.float32),
        jax.ShapeDtypeStruct((N, 1), jnp.float32),
        jax.ShapeDtypeStruct((N, 1), jnp.float32),
        jax.ShapeDtypeStruct((E // D, D), jnp.float32),
        jax.ShapeDtypeStruct((1, 1), jnp.float32),
    ],
)


_mesh = plsc.VectorSubcoreMesh(core_axis_name="c", subcore_axis_name="s")


DC = D // 2


@functools.partial(
    pl.kernel,
    out_type=[
        jax.ShapeDtypeStruct((NC, 2, NP, DC), jnp.float32),
        jax.ShapeDtypeStruct((NC, NP, 16), jnp.float32),
    ],
    mesh=_mesh,
    compiler_params=pltpu.CompilerParams(needs_layout_passes=False,
                                         use_tc_tiling_on_sc=False),
    scratch_types=[
        pltpu.VMEM((NB, BB), jnp.int32),
        pltpu.VMEM((NB, BB), jnp.int32),
        pltpu.VMEM((NB, BB), jnp.float32),
        pltpu.VMEM((N,), jnp.float32),
        pltpu.VMEM((N,), jnp.float32),
        pltpu.VMEM((BB, DC), jnp.float32),
        pltpu.VMEM((BB, DC), jnp.float32),
        pltpu.VMEM((BB, 16), jnp.float32),
        pltpu.VMEM((BB, 16), jnp.float32),
        pltpu.VMEM((128, DC), jnp.float32),
        pltpu.VMEM((128, 16), jnp.float32),
        pltpu.VMEM_SHARED((NP, DC), jnp.float32),
        pltpu.VMEM_SHARED((NP, 16), jnp.float32),
        pltpu.SemaphoreType.DMA,
        pltpu.SemaphoreType.DMA,
        pltpu.SemaphoreType.DMA,
        pltpu.SemaphoreType.DMA,
        pltpu.SemaphoreType.DMA,
        pltpu.SemaphoreType.DMA,
    ],
)
def _sc_edges(h0_hbm, h1_hbm, asrc_hbm, adst_hbm, src_hbm, dst_hbm, ae_hbm,
              msg_out, den_out,
              src_v, dst_v, p_v, asrc_t, adst_t, rows_a, rows_b,
              prows_a, prows_b, zmsg, zden, accm, accd,
              gs_a, gs_b, sm_a, sm_b, sd_a, sd_b):
    cid = lax.axis_index("c")
    sid = lax.axis_index("s")
    wid = sid * NC + cid

    pltpu.sync_copy(src_hbm.at[wid], src_v)
    pltpu.sync_copy(dst_hbm.at[wid], dst_v)
    pltpu.sync_copy(ae_hbm.at[wid], p_v)
    pltpu.sync_copy(asrc_hbm, asrc_t)
    pltpu.sync_copy(adst_hbm, adst_t)

    z16 = jnp.zeros((16,), jnp.float32)

    @pl.loop(0, 128)
    def _zrow(r):
        for c in range(DC // 16):
            zmsg[r, pl.ds(c * 16, 16)] = z16
        zden[r, :] = z16

    for k in range(STRIPE // 128):
        off = sid * STRIPE + k * 128
        pltpu.sync_copy(zmsg, accm.at[pl.ds(off, 128)])
        pltpu.sync_copy(zden, accd.at[pl.ds(off, 128)])

    @pl.loop(0, NB)
    def _alpha(b):
        for k in range(BB // 16):
            sl = pl.ds(k * 16, 16)
            s16 = src_v[b, sl]
            d16 = dst_v[b, sl]
            al = (plsc.load_gather(asrc_t, [s16])
                  + plsc.load_gather(adst_t, [d16])
                  + p_v[b, sl])
            al = jnp.where(al >= 0.0, al, al * NEG)
            p_v[b, sl] = jnp.exp(al)

    plsc.subcore_barrier()

    for half, h_hbm in ((0, h0_hbm), (1, h1_hbm)):
        def _gather(b, buf, gsem):
            pltpu.async_copy(h_hbm.at[src_v.at[b]], buf, gsem)

        def _wait_gather(b, buf, gsem):
            pltpu.make_async_copy(h_hbm.at[src_v.at[b]], buf, gsem).wait()

        def _scale(b, buf, pbuf):
            @pl.loop(0, BB // 16)
            def _grp(g):
                p16 = p_v[b, pl.ds(g * 16, 16)]
                for j in range(16):
                    r = g * 16 + j
                    pb = jnp.full((16,), p16[j], dtype=jnp.float32)
                    if half == 0:
                        pbuf[r, :] = pb
                    for c in range(DC // 16):
                        sl = pl.ds(c * 16, 16)
                        buf[r, sl] = buf[r, sl] * pb

        def _scatter(b, buf, pbuf, smsem, sdsem):
            pltpu.async_copy(buf, accm.at[dst_v.at[b]], smsem, add=True)
            if half == 0:
                pltpu.async_copy(pbuf, accd.at[dst_v.at[b]], sdsem, add=True)

        def _wait_scatter(b, buf, pbuf, smsem, sdsem):
            pltpu.make_async_copy(buf, accm.at[dst_v.at[b]], smsem).wait()
            if half == 0:
                pltpu.make_async_copy(pbuf, accd.at[dst_v.at[b]], sdsem).wait()

        _gather(0, rows_a, gs_a)
        _wait_gather(0, rows_a, gs_a)
        _scale(0, rows_a, prows_a)
        _gather(1, rows_b, gs_b)
        _scatter(0, rows_a, prows_a, sm_a, sd_a)

        @pl.loop(0, (NB - 1) // 2)
        def _pipe(k):
            b1 = 2 * k + 1
            b2 = 2 * k + 2
            _wait_gather(b1, rows_b, gs_b)
            _scale(b1, rows_b, prows_b)
            _wait_scatter(b2 - 2, rows_a, prows_a, sm_a, sd_a)
            _gather(b2, rows_a, gs_a)
            _scatter(b1, rows_b, prows_b, sm_b, sd_b)
            _wait_gather(b2, rows_a, gs_a)
            _scale(b2, rows_a, prows_a)
            _wait_scatter(b1, rows_b, prows_b, sm_b, sd_b)

            @pl.when(b2 + 1 < NB)
            def _next():
                _gather(b2 + 1, rows_b, gs_b)

            _scatter(b2, rows_a, prows_a, sm_a, sd_a)

        _wait_scatter(NB - 1, rows_a, prows_a, sm_a, sd_a)
        plsc.subcore_barrier()

        off = sid * STRIPE
        pltpu.sync_copy(accm.at[pl.ds(off, STRIPE)],
                        msg_out.at[cid, half, pl.ds(off, STRIPE)])
        if half == 0:
            pltpu.sync_copy(accd.at[pl.ds(off, STRIPE)],
                            den_out.at[cid, pl.ds(off, STRIPE)])
            for k in range(STRIPE // 128):
                zoff = sid * STRIPE + k * 128
                pltpu.sync_copy(zmsg, accm.at[pl.ds(zoff, 128)])
            plsc.subcore_barrier()


def _combine_body(mp_ref, dp_ref, h_ref, ansrc_ref, andst_ref, aself_ref,
                  out_ref):
    al = ansrc_ref[...] + andst_ref[...] + aself_ref[...]
    al = jnp.where(al >= 0.0, al, al * NEG)
    ps = jnp.exp(al)
    den = dp_ref[0, :, 0:1] + dp_ref[1, :, 0:1] + ps + 1e-16
    msg = jnp.concatenate(
        [mp_ref[0, 0] + mp_ref[1, 0],
         mp_ref[0, 1] + mp_ref[1, 1]], axis=1)
    out_ref[...] = (msg + ps * h_ref[...]) / den


BN = 1000

_combine_call = pl.pallas_call(
    _combine_body,
    grid=(N // BN,),
    in_specs=[
        pl.BlockSpec((NC, 2, BN, DC), lambda i: (0, 0, i, 0)),
        pl.BlockSpec((NC, BN, 16), lambda i: (0, i, 0)),
        pl.BlockSpec((BN, D), lambda i: (i, 0)),
        pl.BlockSpec((BN, 1), lambda i: (i, 0)),
        pl.BlockSpec((BN, 1), lambda i: (i, 0)),
        pl.BlockSpec((1, 1), lambda i: (0, 0)),
    ],
    out_specs=pl.BlockSpec((BN, D), lambda i: (i, 0)),
    out_shape=jax.ShapeDtypeStruct((N, D), jnp.float32),
)


@jax.jit
def kernel(x, edge_index, edge_attr, W, att_src, att_dst, W_edge, att_edge):
    we = W_edge @ att_edge
    we16 = jnp.pad(we, (0, 5))
    cidx = jnp.arange(16 * D, dtype=jnp.int32)
    m = jnp.zeros((16 * D, D), jnp.float32).at[cidx, cidx // 16].set(we16[cidx % 16])

    ea2 = jnp.pad(edge_attr, ((0, 0), (0, 5))).reshape(E // D, 16 * D)

    h, ansrc, andst, ae2, aes = _dense_call(
        x, W, att_src.reshape(1, D), att_dst.reshape(1, D), ea2, m)

    src = edge_index[0].reshape(NW, NB, BB)
    dst = edge_index[1].reshape(NW, NB, BB)
    ae = ae2.reshape(NW, NB, BB)

    h0 = h[:, :DC]
    h1 = h[:, DC:]
    msg, den = _sc_edges(h0, h1, ansrc.reshape(N), andst.reshape(N),
                         src, dst, ae)

    return _combine_call(msg, den, h, ansrc, andst, aes / E)


# speedup vs baseline: 12.4218x; 1.0211x over previous
import functools

import jax
import jax.numpy as jnp
from jax import lax
from jax.experimental import pallas as pl
from jax.experimental.pallas import tpu as pltpu
from jax.experimental.pallas import tpu_sc as plsc

N = 10000
E = 320000
D = 128
NEG = 0.2

NC = 2
NS = 16
NW = NC * NS
CPT = E // NW
BB = 80
NB = CPT // BB
NP = 10240
STRIPE = NP // NS


def _dense_body(x_ref, w_ref, asrc_ref, adst_ref,
                h_ref, ansrc_ref, andst_ref):
    h = jnp.dot(x_ref[...], w_ref[...], preferred_element_type=jnp.float32)
    h_ref[...] = h
    ansrc_ref[...] = jnp.sum(h * asrc_ref[...], axis=1, keepdims=True)
    andst_ref[...] = jnp.sum(h * adst_ref[...], axis=1, keepdims=True)


_dense_call = pl.pallas_call(
    _dense_body,
    out_shape=[
        jax.ShapeDtypeStruct((N, D), jnp.float32),
        jax.ShapeDtypeStruct((N, 1), jnp.float32),
        jax.ShapeDtypeStruct((N, 1), jnp.float32),
    ],
)


BE = 8000


def _aedge_body(ea_ref, we_ref, ae_ref, aes_ref):
    i = pl.program_id(0)
    ae = jnp.sum(ea_ref[...] * we_ref[...], axis=1, keepdims=True)
    ae_ref[...] = ae

    @pl.when(i == 0)
    def _init():
        aes_ref[...] = jnp.zeros((1, 1), jnp.float32)

    aes_ref[...] += jnp.sum(ae).reshape(1, 1)


_aedge_call = pl.pallas_call(
    _aedge_body,
    grid=(E // BE,),
    in_specs=[
        pl.BlockSpec((BE, 11), lambda i: (i, 0)),
        pl.BlockSpec((1, 11), lambda i: (0, 0)),
    ],
    out_specs=[
        pl.BlockSpec((BE, 1), lambda i: (i, 0)),
        pl.BlockSpec((1, 1), lambda i: (0, 0)),
    ],
    out_shape=[
        jax.ShapeDtypeStruct((E, 1), jnp.float32),
        jax.ShapeDtypeStruct((1, 1), jnp.float32),
    ],
)


_mesh = plsc.VectorSubcoreMesh(core_axis_name="c", subcore_axis_name="s")


DC = D // 2


@functools.partial(
    pl.kernel,
    out_type=[
        jax.ShapeDtypeStruct((NC, 2, NP, DC), jnp.float32),
        jax.ShapeDtypeStruct((NC, NP, 16), jnp.float32),
    ],
    mesh=_mesh,
    compiler_params=pltpu.CompilerParams(needs_layout_passes=False,
                                         use_tc_tiling_on_sc=False),
    scratch_types=[
        pltpu.VMEM((NB, BB), jnp.int32),
        pltpu.VMEM((NB, BB), jnp.int32),
        pltpu.VMEM((NB, BB), jnp.float32),
        pltpu.VMEM((N,), jnp.float32),
        pltpu.VMEM((N,), jnp.float32),
        pltpu.VMEM((BB, DC), jnp.float32),
        pltpu.VMEM((BB, DC), jnp.float32),
        pltpu.VMEM((BB, 16), jnp.float32),
        pltpu.VMEM((BB, 16), jnp.float32),
        pltpu.VMEM((128, DC), jnp.float32),
        pltpu.VMEM((128, 16), jnp.float32),
        pltpu.VMEM_SHARED((NP, DC), jnp.float32),
        pltpu.VMEM_SHARED((NP, 16), jnp.float32),
        pltpu.SemaphoreType.DMA,
        pltpu.SemaphoreType.DMA,
        pltpu.SemaphoreType.DMA,
        pltpu.SemaphoreType.DMA,
        pltpu.SemaphoreType.DMA,
        pltpu.SemaphoreType.DMA,
    ],
)
def _sc_edges(h0_hbm, h1_hbm, asrc_hbm, adst_hbm, src_hbm, dst_hbm, ae_hbm,
              msg_out, den_out,
              src_v, dst_v, p_v, asrc_t, adst_t, rows_a, rows_b,
              prows_a, prows_b, zmsg, zden, accm, accd,
              gs_a, gs_b, sm_a, sm_b, sd_a, sd_b):
    cid = lax.axis_index("c")
    sid = lax.axis_index("s")
    wid = sid * NC + cid

    pltpu.sync_copy(src_hbm.at[wid], src_v)
    pltpu.sync_copy(dst_hbm.at[wid], dst_v)
    pltpu.sync_copy(ae_hbm.at[wid], p_v)
    pltpu.sync_copy(asrc_hbm, asrc_t)
    pltpu.sync_copy(adst_hbm, adst_t)

    z16 = jnp.zeros((16,), jnp.float32)

    @pl.loop(0, 128)
    def _zrow(r):
        for c in range(DC // 16):
            zmsg[r, pl.ds(c * 16, 16)] = z16
        zden[r, :] = z16

    for k in range(STRIPE // 128):
        off = sid * STRIPE + k * 128
        pltpu.sync_copy(zmsg, accm.at[pl.ds(off, 128)])
        pltpu.sync_copy(zden, accd.at[pl.ds(off, 128)])

    @pl.loop(0, NB)
    def _alpha(b):
        for k in range(BB // 16):
            sl = pl.ds(k * 16, 16)
            s16 = src_v[b, sl]
            d16 = dst_v[b, sl]
            al = (plsc.load_gather(asrc_t, [s16])
                  + plsc.load_gather(adst_t, [d16])
                  + p_v[b, sl])
            al = jnp.where(al >= 0.0, al, al * NEG)
            p_v[b, sl] = jnp.exp(al)

    plsc.subcore_barrier()

    for half, h_hbm in ((0, h0_hbm), (1, h1_hbm)):
        def _gather(b, buf, gsem):
            pltpu.async_copy(h_hbm.at[src_v.at[b]], buf, gsem)

        def _wait_gather(b, buf, gsem):
            pltpu.make_async_copy(h_hbm.at[src_v.at[b]], buf, gsem).wait()

        def _scale(b, buf, pbuf):
            @pl.loop(0, BB // 16)
            def _grp(g):
                p16 = p_v[b, pl.ds(g * 16, 16)]
                for j in range(16):
                    r = g * 16 + j
                    pb = jnp.full((16,), p16[j], dtype=jnp.float32)
                    if half == 0:
                        pbuf[r, :] = pb
                    for c in range(DC // 16):
                        sl = pl.ds(c * 16, 16)
                        buf[r, sl] = buf[r, sl] * pb

        def _scatter(b, buf, pbuf, smsem, sdsem):
            pltpu.async_copy(buf, accm.at[dst_v.at[b]], smsem, add=True)
            if half == 0:
                pltpu.async_copy(pbuf, accd.at[dst_v.at[b]], sdsem, add=True)

        def _wait_scatter(b, buf, pbuf, smsem, sdsem):
            pltpu.make_async_copy(buf, accm.at[dst_v.at[b]], smsem).wait()
            if half == 0:
                pltpu.make_async_copy(pbuf, accd.at[dst_v.at[b]], sdsem).wait()

        _gather(0, rows_a, gs_a)
        _wait_gather(0, rows_a, gs_a)
        _scale(0, rows_a, prows_a)
        _gather(1, rows_b, gs_b)
        _scatter(0, rows_a, prows_a, sm_a, sd_a)

        @pl.loop(0, (NB - 1) // 2)
        def _pipe(k):
            b1 = 2 * k + 1
            b2 = 2 * k + 2
            _wait_gather(b1, rows_b, gs_b)
            _scale(b1, rows_b, prows_b)
            _wait_scatter(b2 - 2, rows_a, prows_a, sm_a, sd_a)
            _gather(b2, rows_a, gs_a)
            _scatter(b1, rows_b, prows_b, sm_b, sd_b)
            _wait_gather(b2, rows_a, gs_a)
            _scale(b2, rows_a, prows_a)
            _wait_scatter(b1, rows_b, prows_b, sm_b, sd_b)

            @pl.when(b2 + 1 < NB)
            def _next():
                _gather(b2 + 1, rows_b, gs_b)

            _scatter(b2, rows_a, prows_a, sm_a, sd_a)

        _wait_scatter(NB - 1, rows_a, prows_a, sm_a, sd_a)
        plsc.subcore_barrier()

        off = sid * STRIPE
        pltpu.sync_copy(accm.at[pl.ds(off, STRIPE)],
                        msg_out.at[cid, half, pl.ds(off, STRIPE)])
        if half == 0:
            pltpu.sync_copy(accd.at[pl.ds(off, STRIPE)],
                            den_out.at[cid, pl.ds(off, STRIPE)])
            for k in range(STRIPE // 128):
                zoff = sid * STRIPE + k * 128
                pltpu.sync_copy(zmsg, accm.at[pl.ds(zoff, 128)])
            plsc.subcore_barrier()


def _combine_body(mp_ref, dp_ref, h_ref, ansrc_ref, andst_ref, aself_ref,
                  out_ref):
    al = ansrc_ref[...] + andst_ref[...] + aself_ref[...]
    al = jnp.where(al >= 0.0, al, al * NEG)
    ps = jnp.exp(al)
    den = dp_ref[0, :, 0:1] + dp_ref[1, :, 0:1] + ps + 1e-16
    msg = jnp.concatenate(
        [mp_ref[0, 0] + mp_ref[1, 0],
         mp_ref[0, 1] + mp_ref[1, 1]], axis=1)
    out_ref[...] = (msg + ps * h_ref[...]) / den


BN = 1000

_combine_call = pl.pallas_call(
    _combine_body,
    grid=(N // BN,),
    in_specs=[
        pl.BlockSpec((NC, 2, BN, DC), lambda i: (0, 0, i, 0)),
        pl.BlockSpec((NC, BN, 16), lambda i: (0, i, 0)),
        pl.BlockSpec((BN, D), lambda i: (i, 0)),
        pl.BlockSpec((BN, 1), lambda i: (i, 0)),
        pl.BlockSpec((BN, 1), lambda i: (i, 0)),
        pl.BlockSpec((1, 1), lambda i: (0, 0)),
    ],
    out_specs=pl.BlockSpec((BN, D), lambda i: (i, 0)),
    out_shape=jax.ShapeDtypeStruct((N, D), jnp.float32),
)


@jax.jit
def kernel(x, edge_index, edge_attr, W, att_src, att_dst, W_edge, att_edge):
    we = W_edge @ att_edge

    h, ansrc, andst = _dense_call(
        x, W, att_src.reshape(1, D), att_dst.reshape(1, D))
    ae2, aes = _aedge_call(edge_attr, we.reshape(1, 11))

    src = edge_index[0].reshape(NW, NB, BB)
    dst = edge_index[1].reshape(NW, NB, BB)
    ae = ae2.reshape(NW, NB, BB)

    h0 = h[:, :DC]
    h1 = h[:, DC:]
    msg, den = _sc_edges(h0, h1, ansrc.reshape(N), andst.reshape(N),
                         src, dst, ae)

    return _combine_call(msg, den, h, ansrc, andst, aes / E)


# speedup vs baseline: 13.5548x; 1.0912x over previous
import functools

import jax
import jax.numpy as jnp
from jax import lax
from jax.experimental import pallas as pl
from jax.experimental.pallas import tpu as pltpu
from jax.experimental.pallas import tpu_sc as plsc

N = 10000
E = 320000
D = 128
NEG = 0.2

NC = 2
NS = 16
NW = NC * NS
CPT = E // NW
BB = 80
NB = CPT // BB
NP = 10240
STRIPE = NP // NS


def _dense_body(x_ref, w_ref, asrc_ref, adst_ref,
                h_ref, ansrc_ref, andst_ref):
    h = jnp.dot(x_ref[...], w_ref[...], preferred_element_type=jnp.float32)
    h_ref[...] = h
    ansrc_ref[...] = jnp.sum(h * asrc_ref[...], axis=1, keepdims=True)
    andst_ref[...] = jnp.sum(h * adst_ref[...], axis=1, keepdims=True)


_dense_call = pl.pallas_call(
    _dense_body,
    out_shape=[
        jax.ShapeDtypeStruct((N, D), jnp.float32),
        jax.ShapeDtypeStruct((N, 1), jnp.float32),
        jax.ShapeDtypeStruct((N, 1), jnp.float32),
    ],
)


SLE = 400
NSL = CPT // SLE
NBS = SLE // BB

_mesh = plsc.VectorSubcoreMesh(core_axis_name="c", subcore_axis_name="s")


DC = D // 2


@functools.partial(
    pl.kernel,
    out_type=[
        jax.ShapeDtypeStruct((NC, 2, NP, DC), jnp.float32),
        jax.ShapeDtypeStruct((NW, N), jnp.float32),
        jax.ShapeDtypeStruct((NW, 16), jnp.float32),
    ],
    mesh=_mesh,
    compiler_params=pltpu.CompilerParams(needs_layout_passes=False,
                                         use_tc_tiling_on_sc=False),
    scratch_types=[
        pltpu.VMEM((NB, BB), jnp.int32),
        pltpu.VMEM((NB, BB), jnp.int32),
        pltpu.VMEM((NB, BB), jnp.float32),
        pltpu.VMEM((SLE * 11,), jnp.float32),
        pltpu.VMEM((11, 16), jnp.float32),
        pltpu.VMEM((16,), jnp.float32),
        pltpu.VMEM((N,), jnp.float32),
        pltpu.VMEM((N,), jnp.float32),
        pltpu.VMEM((N,), jnp.float32),
        pltpu.VMEM((BB, DC), jnp.float32),
        pltpu.VMEM((BB, DC), jnp.float32),
        pltpu.VMEM((64, DC), jnp.float32),
        pltpu.VMEM_SHARED((NP, DC), jnp.float32),
        pltpu.SemaphoreType.DMA,
        pltpu.SemaphoreType.DMA,
        pltpu.SemaphoreType.DMA,
        pltpu.SemaphoreType.DMA,
    ],
)
def _sc_edges(h0_hbm, h1_hbm, asrc_hbm, adst_hbm, src_hbm, dst_hbm,
              ea_hbm, web_hbm,
              msg_out, den_out, aes_out,
              src_v, dst_v, p_v, ea_v, web_v, asum_v, asrc_t, adst_t, den_t,
              rows_a, rows_b, zmsg, accm,
              gs_a, gs_b, sm_a, sm_b):
    cid = lax.axis_index("c")
    sid = lax.axis_index("s")
    wid = sid * NC + cid

    pltpu.sync_copy(src_hbm.at[wid], src_v)
    pltpu.sync_copy(dst_hbm.at[wid], dst_v)
    pltpu.sync_copy(asrc_hbm, asrc_t)
    pltpu.sync_copy(adst_hbm, adst_t)
    pltpu.sync_copy(web_hbm, web_v)

    z16 = jnp.zeros((16,), jnp.float32)
    asum_v[:] = z16

    @pl.loop(0, N // 16)
    def _zden(i):
        den_t[pl.ds(i * 16, 16)] = z16

    @pl.loop(0, 64)
    def _zrow(r):
        for c in range(DC // 16):
            zmsg[r, pl.ds(c * 16, 16)] = z16

    for k in range(STRIPE // 64):
        off = sid * STRIPE + k * 64
        pltpu.sync_copy(zmsg, accm.at[pl.ds(off, 64)])

    lanes = lax.iota(jnp.int32, 16)
    for s in range(NSL):
        pltpu.sync_copy(ea_hbm.at[pl.ds((wid * CPT + s * SLE) * 11, SLE * 11)],
                        ea_v)

        @pl.loop(0, NBS)
        def _alpha(b2):
            b = s * NBS + b2
            for k in range(BB // 16):
                sl = pl.ds(k * 16, 16)
                ebase = (lanes + (b2 * BB + k * 16)) * 11
                ae16 = jnp.zeros((16,), jnp.float32)
                for j in range(11):
                    ae16 = ae16 + plsc.load_gather(ea_v, [ebase + j]) * web_v[j, :]
                asum_v[:] = asum_v[:] + ae16
                s16 = src_v[b, sl]
                d16 = dst_v[b, sl]
                al = (plsc.load_gather(asrc_t, [s16])
                      + plsc.load_gather(adst_t, [d16])
                      + ae16)
                al = jnp.where(al >= 0.0, al, al * NEG)
                pe = jnp.exp(al)
                p_v[b, sl] = pe
                plsc.addupdate_scatter(den_t, [d16], pe)

    pltpu.sync_copy(asum_v, aes_out.at[wid])
    pltpu.sync_copy(den_t, den_out.at[wid])
    plsc.subcore_barrier()

    for half, h_hbm in ((0, h0_hbm), (1, h1_hbm)):
        def _gather(b, buf, gsem):
            pltpu.async_copy(h_hbm.at[src_v.at[b]], buf, gsem)

        def _wait_gather(b, buf, gsem):
            pltpu.make_async_copy(h_hbm.at[src_v.at[b]], buf, gsem).wait()

        def _scale(b, buf):
            @pl.loop(0, BB // 16)
            def _grp(g):
                p16 = p_v[b, pl.ds(g * 16, 16)]
                for j in range(16):
                    r = g * 16 + j
                    pb = jnp.full((16,), p16[j], dtype=jnp.float32)
                    for c in range(DC // 16):
                        sl = pl.ds(c * 16, 16)
                        buf[r, sl] = buf[r, sl] * pb

        def _scatter(b, buf, smsem):
            pltpu.async_copy(buf, accm.at[dst_v.at[b]], smsem, add=True)

        def _wait_scatter(b, buf, smsem):
            pltpu.make_async_copy(buf, accm.at[dst_v.at[b]], smsem).wait()

        _gather(0, rows_a, gs_a)
        _wait_gather(0, rows_a, gs_a)
        _scale(0, rows_a)
        _gather(1, rows_b, gs_b)
        _scatter(0, rows_a, sm_a)

        @pl.loop(0, (NB - 1) // 2)
        def _pipe(k):
            b1 = 2 * k + 1
            b2 = 2 * k + 2
            _wait_gather(b1, rows_b, gs_b)
            _scale(b1, rows_b)
            _wait_scatter(b2 - 2, rows_a, sm_a)
            _gather(b2, rows_a, gs_a)
            _scatter(b1, rows_b, sm_b)
            _wait_gather(b2, rows_a, gs_a)
            _scale(b2, rows_a)
            _wait_scatter(b1, rows_b, sm_b)

            @pl.when(b2 + 1 < NB)
            def _next():
                _gather(b2 + 1, rows_b, gs_b)

            _scatter(b2, rows_a, sm_a)

        _wait_scatter(NB - 1, rows_a, sm_a)
        plsc.subcore_barrier()

        off = sid * STRIPE
        pltpu.sync_copy(accm.at[pl.ds(off, STRIPE)],
                        msg_out.at[cid, half, pl.ds(off, STRIPE)])
        if half == 0:
            for k in range(STRIPE // 64):
                zoff = sid * STRIPE + k * 64
                pltpu.sync_copy(zmsg, accm.at[pl.ds(zoff, 64)])
            plsc.subcore_barrier()


def _combine_body(mp_ref, dp_ref, h_ref, ansrc_ref, andst_ref, aes_ref,
                  out_ref):
    a_self = (jnp.sum(aes_ref[...]) / E).reshape(1, 1)
    al = ansrc_ref[...] + andst_ref[...] + a_self
    al = jnp.where(al >= 0.0, al, al * NEG)
    ps = jnp.exp(al)
    den = jnp.sum(dp_ref[...], axis=1, keepdims=True) + ps + 1e-16
    msg = jnp.concatenate(
        [mp_ref[0, 0] + mp_ref[1, 0],
         mp_ref[0, 1] + mp_ref[1, 1]], axis=1)
    out_ref[...] = (msg + ps * h_ref[...]) / den


BN = 1000

_combine_call = pl.pallas_call(
    _combine_body,
    grid=(N // BN,),
    in_specs=[
        pl.BlockSpec((NC, 2, BN, DC), lambda i: (0, 0, i, 0)),
        pl.BlockSpec((BN, NW), lambda i: (i, 0)),
        pl.BlockSpec((BN, D), lambda i: (i, 0)),
        pl.BlockSpec((BN, 1), lambda i: (i, 0)),
        pl.BlockSpec((BN, 1), lambda i: (i, 0)),
        pl.BlockSpec((NW, 16), lambda i: (0, 0)),
    ],
    out_specs=pl.BlockSpec((BN, D), lambda i: (i, 0)),
    out_shape=jax.ShapeDtypeStruct((N, D), jnp.float32),
)


@jax.jit
def kernel(x, edge_index, edge_attr, W, att_src, att_dst, W_edge, att_edge):
    we = W_edge @ att_edge
    web = jnp.broadcast_to(we[:, None], (11, 16))

    h, ansrc, andst = _dense_call(
        x, W, att_src.reshape(1, D), att_dst.reshape(1, D))

    src = edge_index[0].reshape(NW, NB, BB)
    dst = edge_index[1].reshape(NW, NB, BB)

    h0 = h[:, :DC]
    h1 = h[:, DC:]
    msg, den, aes = _sc_edges(h0, h1, ansrc.reshape(N), andst.reshape(N),
                              src, dst, edge_attr.reshape(E * 11), web)

    return _combine_call(msg, den.T, h, ansrc, andst, aes)
